# Initial kernel scaffold; baseline (speedup 1.0000x reference)
import jax
import jax.numpy as jnp
from jax.experimental import pallas as pl


def kernel(x):
    raise NotImplementedError("write your pallas kernel here")


# speedup vs baseline: 10.1408x; 10.1408x over previous
import math

import jax
import jax.numpy as jnp
from jax import lax
from jax.experimental import pallas as pl

_SPARSITY = 0.05
_BLOCK_M = 8


def _order_key_u32(x):
    b = lax.bitcast_convert_type(x, jnp.int32)
    flip = lax.shift_right_arithmetic(b, 31) | jnp.int32(-2147483648)
    return lax.bitcast_convert_type(b ^ flip, jnp.uint32)


def _key_to_f32(u):
    ui = lax.bitcast_convert_type(u, jnp.int32)
    flip = ~lax.shift_right_arithmetic(ui, 31) | jnp.int32(-2147483648)
    return lax.bitcast_convert_type(ui ^ flip, jnp.float32)


def _kwta_block(x_ref, o_ref, *, k):
    x = x_ref[...]
    ukey = _order_key_u32(x)
    m = x.shape[0]
    zero = jnp.zeros((m, 1), jnp.uint32)

    def body(i, carry):
        t1, t2 = carry
        bit = jnp.uint32(31) - jnp.uint32(i)
        add = lax.shift_left(jnp.uint32(1), bit)
        c1 = t1 | add
        c2 = t2 | add
        n1 = jnp.sum((ukey >= c1).astype(jnp.int32), axis=1, keepdims=True)
        n2 = jnp.sum((ukey >= c2).astype(jnp.int32), axis=1, keepdims=True)
        t1 = jnp.where(n1 >= k, c1, t1)
        t2 = jnp.where(n2 >= k + 1, c2, t2)
        return t1, t2

    t1, t2 = lax.fori_loop(0, 32, body, (zero, zero))
    thr = (_key_to_f32(t1) + _key_to_f32(t2)) * jnp.float32(0.5)
    o_ref[...] = (x > thr).astype(jnp.float32)


def kernel(x):
    m, n = x.shape
    k = math.ceil(_SPARSITY * n)
    grid = (m // _BLOCK_M,)
    return pl.pallas_call(
        lambda x_ref, o_ref: _kwta_block(x_ref, o_ref, k=k),
        grid=grid,
        in_specs=[pl.BlockSpec((_BLOCK_M, n), lambda i: (i, 0))],
        out_specs=pl.BlockSpec((_BLOCK_M, n), lambda i: (i, 0)),
        out_shape=jax.ShapeDtypeStruct((m, n), jnp.float32),
    )(x)
